# baseline (speedup 1.0000x reference)
import functools

import jax
import jax.numpy as jnp
from jax import lax
from jax.experimental import pallas as pl
from jax.experimental.pallas import tpu as pltpu
from jax.experimental.pallas import tpu_sc as plsc


def _make_sc_gather(B, G, D2):
    info = plsc.get_sparse_core_info()
    NC, NS, L = info.num_cores, info.num_subcores, info.num_lanes
    NW = NC * NS
    assert B % (8 * NW) == 0
    b_per_w = B // NW
    mesh = plsc.VectorSubcoreMesh(core_axis_name="c", subcore_axis_name="s")
    D = D2 // 2

    @functools.partial(
        pl.kernel,
        mesh=mesh,
        out_type=jax.ShapeDtypeStruct((B, D2), jnp.float32),
        scratch_types=[
            pltpu.VMEM((b_per_w,), jnp.int32),
            pltpu.VMEM((b_per_w,), jnp.int32),
            pltpu.VMEM((b_per_w, D2), jnp.float32),
            pltpu.SemaphoreType.DMA,
        ],
        compiler_params=pltpu.CompilerParams(needs_layout_passes=False),
    )
    def gather(ids_hbm, lin_hbm, out_hbm, idx_v, pair_v, rows_v, sem):
        wid = lax.axis_index("s") * NC + lax.axis_index("c")
        base = wid * b_per_w
        pltpu.sync_copy(ids_hbm.at[pl.ds(base, b_per_w)], idx_v)

        def compute_pairs(i, carry):
            v = idx_v[pl.ds(i * L, L)]
            pair_v[pl.ds(i * L, L)] = lax.shift_right_logical(v, 1)
            return carry

        lax.fori_loop(0, b_per_w // L, compute_pairs, 0)
        pltpu.async_copy(lin_hbm.at[pair_v], rows_v, sem).wait()

        def extract(g, carry):
            v = idx_v[pl.ds(g * L, L)]
            for l in range(L):
                s = v[l]

                @pl.when((s & 1) == 1)
                def _():
                    i = g * L + l
                    for q in range(D // L):
                        rows_v[i, pl.ds(q * L, L)] = rows_v[
                            i, pl.ds(D + q * L, L)
                        ]

            return carry

        lax.fori_loop(0, b_per_w // L, extract, 0)
        pltpu.sync_copy(rows_v, out_hbm.at[pl.ds(base, b_per_w)])

    return gather


def kernel(item_ids, emb_weight):
    B, = item_ids.shape
    V, D = emb_weight.shape
    ids = item_ids.astype(jnp.int32)
    h = V // 2
    lin = jnp.concatenate(
        [
            emb_weight[:h].reshape(h // 2, 2 * D),
            emb_weight[h:].reshape(h // 2, 2 * D),
        ],
        axis=0,
    )
    wide = _make_sc_gather(B, V // 2, 2 * D)(ids, lin)
    return wide[:, :D]


# speedup vs baseline: 2.9347x; 2.9347x over previous
import functools

import jax
import jax.numpy as jnp
from jax import lax
from jax.experimental import pallas as pl
from jax.experimental.pallas import tpu as pltpu
from jax.experimental.pallas import tpu_sc as plsc


def _make_gather(B, V, D):
    info = plsc.get_sparse_core_info()
    NC, NS, L = info.num_cores, info.num_subcores, info.num_lanes
    NW = NC * NS
    assert B % (8 * NW) == 0 and (B // NW) % L == 0
    b_per_w = B // NW
    mesh = plsc.VectorSubcoreMesh(core_axis_name="c", subcore_axis_name="s")

    @functools.partial(
        pl.kernel,
        mesh=mesh,
        out_type=jax.ShapeDtypeStruct((B, D), jnp.float32),
        scratch_types=[
            pltpu.VMEM((b_per_w,), jnp.int32),
            pltpu.VMEM((b_per_w, D), jnp.float32),
            pltpu.SemaphoreType.DMA,
        ],
        compiler_params=pltpu.CompilerParams(needs_layout_passes=False),
    )
    def gather(ids_hbm, table_hbm, out_hbm, idx_v, rows_v, sem):
        wid = lax.axis_index("s") * NC + lax.axis_index("c")
        base = wid * b_per_w
        pltpu.sync_copy(ids_hbm.at[pl.ds(base, b_per_w)], idx_v)

        def fetch_group(g, carry):
            v = idx_v[pl.ds(g * L, L)]
            for l in range(L):
                pltpu.async_copy(
                    table_hbm.at[pl.ds(v[l], 1)],
                    rows_v.at[pl.ds(g * L + l, 1)],
                    sem,
                )
            return carry

        lax.fori_loop(0, b_per_w // L, fetch_group, 0)
        pltpu.make_async_copy(
            table_hbm.at[pl.ds(0, b_per_w)], rows_v, sem
        ).wait()
        pltpu.sync_copy(rows_v, out_hbm.at[pl.ds(base, b_per_w)])

    return gather


def kernel(item_ids, emb_weight):
    B, = item_ids.shape
    V, D = emb_weight.shape
    ids = item_ids.astype(jnp.int32)
    return _make_gather(B, V, D)(ids, emb_weight)
